# Initial kernel scaffold; baseline (speedup 1.0000x reference)
import jax
import jax.numpy as jnp
from jax.experimental import pallas as pl


def kernel(x, edge_index, W1, b1, W2, b2, W3, b3):
    raise NotImplementedError("write your pallas kernel here")


# speedup vs baseline: 116.6183x; 116.6183x over previous
import functools

import jax
import jax.numpy as jnp
from jax import lax
from jax.experimental import pallas as pl
from jax.experimental.pallas import tpu as pltpu
from jax.experimental.pallas import tpu_sc as plsc

N = 100000
NP = 100096
NPL = NP // 8
E = 6400000
NC, NS = 2, 16
NW = NC * NS
K = 2048
WINDOWS = 100
EPW = K * WINDOWS
EP = EPW * NW
NPT = NP // NS

_mesh = plsc.VectorSubcoreMesh(core_axis_name="c", subcore_axis_name="s")


@functools.partial(
    pl.kernel,
    out_type=jax.ShapeDtypeStruct((NC * NP,), jnp.float32),
    mesh=_mesh,
    scratch_types=[
        pltpu.VMEM_SHARED((NP,), jnp.float32),
        pltpu.VMEM((K,), jnp.int32),
        pltpu.VMEM((K,), jnp.float32),
        pltpu.VMEM((NPT,), jnp.float32),
    ],
)
def _deg_sc(dst_hbm, ones_hbm, zeros_hbm, out_hbm, deg_sh, dst_v, ones_v,
            stage_v):
    cid = lax.axis_index("c")
    sid = lax.axis_index("s")
    wid = sid * NC + cid
    pltpu.sync_copy(zeros_hbm.at[pl.ds(sid * NPT, NPT)], stage_v)
    pltpu.sync_copy(stage_v, deg_sh.at[pl.ds(sid * NPT, NPT)])
    pltpu.sync_copy(ones_hbm, ones_v)
    plsc.subcore_barrier()
    e0 = wid * EPW

    def body(w, carry):
        pltpu.sync_copy(dst_hbm.at[pl.ds(e0 + w * K, K)], dst_v)
        pltpu.sync_copy(ones_v, deg_sh.at[dst_v], add=True)
        return carry

    lax.fori_loop(0, WINDOWS, body, 0)
    plsc.subcore_barrier()
    pltpu.sync_copy(deg_sh.at[pl.ds(sid * NPT, NPT)], stage_v)
    pltpu.sync_copy(stage_v, out_hbm.at[pl.ds(cid * NP + sid * NPT, NPT)])


@functools.partial(
    pl.kernel,
    out_type=jax.ShapeDtypeStruct((NC * NP,), jnp.float32),
    mesh=_mesh,
    scratch_types=[
        pltpu.VMEM_SHARED((NP,), jnp.float32),
        pltpu.VMEM_SHARED((NP,), jnp.float32),
        pltpu.VMEM((K,), jnp.int32),
        pltpu.VMEM((K,), jnp.int32),
        pltpu.VMEM((K,), jnp.float32),
        pltpu.VMEM((NPT,), jnp.float32),
        pltpu.SemaphoreType.DMA,
    ],
)
def _prop1_sc(u_hbm, src_hbm, dst_hbm, zeros_hbm, out_hbm,
              u_sh, acc_sh, src_v, dst_v, msg_v, stage_v, sem):
    cid = lax.axis_index("c")
    sid = lax.axis_index("s")
    wid = sid * NC + cid
    ns = pl.ds(sid * NPT, NPT)
    pltpu.sync_copy(zeros_hbm.at[ns], stage_v)
    pltpu.sync_copy(stage_v, acc_sh.at[ns])
    pltpu.sync_copy(u_hbm.at[ns], stage_v)
    pltpu.sync_copy(stage_v, u_sh.at[ns])
    plsc.subcore_barrier()
    e0 = wid * EPW

    def body(w, carry):
        pltpu.sync_copy(src_hbm.at[pl.ds(e0 + w * K, K)], src_v)
        pltpu.sync_copy(dst_hbm.at[pl.ds(e0 + w * K, K)], dst_v)
        pltpu.async_copy(u_sh.at[src_v], msg_v, sem).wait()
        pltpu.sync_copy(msg_v, acc_sh.at[dst_v], add=True)
        return carry

    lax.fori_loop(0, WINDOWS, body, 0)
    plsc.subcore_barrier()
    pltpu.sync_copy(acc_sh.at[ns], stage_v)
    pltpu.sync_copy(stage_v, out_hbm.at[pl.ds(cid * NP + sid * NPT, NPT)])


@functools.partial(
    pl.kernel,
    out_type=[jax.ShapeDtypeStruct((NC * NP,), jnp.float32)] * 3,
    mesh=_mesh,
    scratch_types=(
        [pltpu.VMEM_SHARED((NP,), jnp.float32)] * 3
        + [pltpu.VMEM_SHARED((NP,), jnp.float32)] * 3
        + [
            pltpu.VMEM((K,), jnp.int32),
            pltpu.VMEM((K,), jnp.int32),
        ]
        + [pltpu.VMEM((K,), jnp.float32)] * 3
        + [
            pltpu.VMEM((NPT,), jnp.float32),
            pltpu.SemaphoreType.DMA,
        ]
    ),
)
def _prop3_sc(u0_hbm, u1_hbm, u2_hbm, src_hbm, dst_hbm, zeros_hbm,
              o0_hbm, o1_hbm, o2_hbm,
              u_sh0, u_sh1, u_sh2, a_sh0, a_sh1, a_sh2,
              src_v, dst_v, m0_v, m1_v, m2_v, stage_v, sem):
    cid = lax.axis_index("c")
    sid = lax.axis_index("s")
    wid = sid * NC + cid
    ns = pl.ds(sid * NPT, NPT)
    u_shs = (u_sh0, u_sh1, u_sh2)
    a_shs = (a_sh0, a_sh1, a_sh2)
    msgs = (m0_v, m1_v, m2_v)
    pltpu.sync_copy(zeros_hbm.at[ns], stage_v)
    for a_sh in a_shs:
        pltpu.sync_copy(stage_v, a_sh.at[ns])
    for u_hbm, u_sh in zip((u0_hbm, u1_hbm, u2_hbm), u_shs):
        pltpu.sync_copy(u_hbm.at[ns], stage_v)
        pltpu.sync_copy(stage_v, u_sh.at[ns])
    plsc.subcore_barrier()
    e0 = wid * EPW

    def body(w, carry):
        pltpu.sync_copy(src_hbm.at[pl.ds(e0 + w * K, K)], src_v)
        pltpu.sync_copy(dst_hbm.at[pl.ds(e0 + w * K, K)], dst_v)
        cps = [pltpu.async_copy(u_sh.at[src_v], m_v, sem)
               for u_sh, m_v in zip(u_shs, msgs)]
        for cp in cps:
            cp.wait()
        for a_sh, m_v in zip(a_shs, msgs):
            pltpu.sync_copy(m_v, a_sh.at[dst_v], add=True)
        return carry

    lax.fori_loop(0, WINDOWS, body, 0)
    plsc.subcore_barrier()
    for a_sh, o_hbm in zip(a_shs, (o0_hbm, o1_hbm, o2_hbm)):
        pltpu.sync_copy(a_sh.at[ns], stage_v)
        pltpu.sync_copy(stage_v, o_hbm.at[pl.ds(cid * NP + sid * NPT, NPT)])


def _prep_body(degp_ref, xt_ref, dinv_ref, u1_ref):
    deg = degp_ref[0] + degp_ref[1] + 1.0
    dinv = lax.rsqrt(deg)
    dinv_ref[...] = dinv
    u1_ref[...] = xt_ref[...] * dinv


_prep_tc = pl.pallas_call(
    _prep_body,
    out_shape=[
        jax.ShapeDtypeStruct((8, NPL), jnp.float32),
        jax.ShapeDtypeStruct((8, NPL), jnp.float32),
    ],
)


def _epi_body(fi, fo, last, p_ref, u_ref, dinv_ref, w_ref, b_ref, *outs):
    dinv = dinv_ref[...]
    t = [(p_ref[0, k] + p_ref[1, k] + u_ref[k]) * dinv for k in range(fi)]
    for j in range(fo):
        s = t[0] * w_ref[0, j]
        for k in range(1, fi):
            s = s + t[k] * w_ref[k, j]
        h = jnp.maximum(s + b_ref[j], 0.0)
        outs[0][j] = h
        if not last:
            outs[1][j] = h * dinv


def _make_epi(fi, fo, last):
    outs = [jax.ShapeDtypeStruct((fo, 8, NPL), jnp.float32)]
    if not last:
        outs.append(jax.ShapeDtypeStruct((fo, 8, NPL), jnp.float32))
    return pl.pallas_call(
        functools.partial(_epi_body, fi, fo, last),
        in_specs=[
            pl.BlockSpec(memory_space=pltpu.VMEM),
            pl.BlockSpec(memory_space=pltpu.VMEM),
            pl.BlockSpec(memory_space=pltpu.VMEM),
            pl.BlockSpec(memory_space=pltpu.SMEM),
            pl.BlockSpec(memory_space=pltpu.SMEM),
        ],
        out_shape=outs,
    )


_epi1 = _make_epi(1, 3, last=False)
_epi2 = _make_epi(3, 3, last=False)
_epi3 = _make_epi(3, 3, last=True)


def kernel(x, edge_index, W1, b1, W2, b2, W3, b3):
    src = edge_index[0].astype(jnp.int32)
    dst = edge_index[1].astype(jnp.int32)
    pad = N + (jnp.arange(EP - E, dtype=jnp.int32) % (NP - N))
    srcp = jnp.concatenate([src, pad])
    dstp = jnp.concatenate([dst, pad])
    zeros1 = jnp.zeros((NP,), jnp.float32)
    ones = jnp.ones((K,), jnp.float32)
    xt = jnp.pad(x[:, 0], (0, NP - N)).reshape(8, NPL)

    degp = _deg_sc(dstp, ones, zeros1)
    dinvt, u1t = _prep_tc(degp.reshape(NC, 8, NPL), xt)

    p1 = _prop1_sc(u1t.reshape(NP), srcp, dstp, zeros1)
    h1t, u2t = _epi1(p1.reshape(NC, 1, 8, NPL), u1t.reshape(1, 8, NPL),
                     dinvt, W1, b1)

    u2c = u2t.reshape(3, NP)
    p2 = _prop3_sc(u2c[0], u2c[1], u2c[2], srcp, dstp, zeros1)
    p2s = jnp.stack([c.reshape(NC, 8, NPL) for c in p2], axis=1)
    h2t, u3t = _epi2(p2s, u2t, dinvt, W2, b2)

    u3c = u3t.reshape(3, NP)
    p3 = _prop3_sc(u3c[0], u3c[1], u3c[2], srcp, dstp, zeros1)
    p3s = jnp.stack([c.reshape(NC, 8, NPL) for c in p3], axis=1)
    (h3t,) = _epi3(p3s, u3t, dinvt, W3, b3)

    return h3t.reshape(3, NP).T[:N]
